# baseline (speedup 1.0000x reference)
import functools

import jax
import jax.numpy as jnp
from jax import lax
from jax.experimental import pallas as pl
from jax.experimental.pallas import tpu as pltpu
from jax.experimental.pallas import tpu_sc as plsc

_BATCH = 16384
_EMB = 64
_L = 16
_NC = 2
_NS = 16
_NW = _NC * _NS
_N_PER = _BATCH // _NW


def _sc_body(u_hbm, v_hbm, table_hbm, out_hbm,
             uidx_v, vidx_v, ublk, vblk, out_v, sem_u, sem_v):
    wid = lax.axis_index("s") * _NC + lax.axis_index("c")
    base = wid * _N_PER

    pltpu.sync_copy(u_hbm.at[pl.ds(base, _N_PER)], uidx_v)
    pltpu.sync_copy(v_hbm.at[pl.ds(base, _N_PER)], vidx_v)

    lane = lax.iota(jnp.int32, _L)

    def fire(g, b):
        sl = pl.ds(g * _L, _L)
        ubase = jnp.bitwise_and(uidx_v[sl], ~7)
        vbase = jnp.bitwise_and(vidx_v[sl], ~7)
        for k in range(_L):
            ub = pl.multiple_of(ubase[k], 8)
            vb = pl.multiple_of(vbase[k], 8)
            pltpu.async_copy(table_hbm.at[pl.ds(ub, 8), :],
                             ublk.at[b, k], sem_u.at[b])
            pltpu.async_copy(table_hbm.at[pl.ds(vb, 8), :],
                             vblk.at[b, k], sem_v.at[b])

    def drain(b):
        for k in range(_L):
            pltpu.make_async_copy(table_hbm.at[pl.ds(0, 8), :],
                                  ublk.at[b, k], sem_u.at[b]).wait()
            pltpu.make_async_copy(table_hbm.at[pl.ds(0, 8), :],
                                  vblk.at[b, k], sem_v.at[b]).wait()

    def compute(g, b):
        sl = pl.ds(g * _L, _L)
        uoff = jnp.bitwise_and(uidx_v[sl], 7)
        voff = jnp.bitwise_and(vidx_v[sl], 7)
        acc = jnp.zeros((_L,), jnp.float32)
        for j in range(_EMB):
            jc = jnp.full((_L,), j, jnp.int32)
            uu = plsc.load_gather(ublk.at[b], [lane, uoff, jc])
            vv = plsc.load_gather(vblk.at[b], [lane, voff, jc])
            acc = acc + uu * vv
        out_v[sl] = 1.0 / (1.0 + jnp.exp(-acc))

    ngroups = _N_PER // _L
    fire(jnp.int32(0), jnp.int32(0))

    def gbody(g, carry):
        b = jnp.bitwise_and(g, 1)

        @pl.when(g + 1 < ngroups)
        def _():
            fire(g + 1, 1 - b)

        drain(b)
        compute(g, b)
        return carry

    lax.fori_loop(0, ngroups, gbody, 0)

    pltpu.sync_copy(out_v, out_hbm.at[pl.ds(base, _N_PER)])


@jax.jit
def _dist_mult(u, v, emb_weight):
    mesh = plsc.VectorSubcoreMesh(
        core_axis_name="c", subcore_axis_name="s",
        num_cores=_NC, num_subcores=_NS)
    run = pl.kernel(
        _sc_body,
        out_type=jax.ShapeDtypeStruct((_BATCH,), jnp.float32),
        mesh=mesh,
        scratch_types=[
            pltpu.VMEM((_N_PER,), jnp.int32),
            pltpu.VMEM((_N_PER,), jnp.int32),
            pltpu.VMEM((2, _L, 8, _EMB), jnp.float32),
            pltpu.VMEM((2, _L, 8, _EMB), jnp.float32),
            pltpu.VMEM((_N_PER,), jnp.float32),
            pltpu.SemaphoreType.DMA((2,)),
            pltpu.SemaphoreType.DMA((2,)),
        ],
        compiler_params=pltpu.CompilerParams(
            needs_layout_passes=False, use_tc_tiling_on_sc=True),
    )
    return run(u, v, emb_weight)


def kernel(u, v, emb_weight):
    return _dist_mult(u.astype(jnp.int32), v.astype(jnp.int32), emb_weight)


# speedup vs baseline: 1.0187x; 1.0187x over previous
import functools

import jax
import jax.numpy as jnp
from jax import lax
from jax.experimental import pallas as pl
from jax.experimental.pallas import tpu as pltpu
from jax.experimental.pallas import tpu_sc as plsc

_BATCH = 16384
_EMB = 64
_L = 16
_NC = 2
_NS = 16
_NW = _NC * _NS
_N_PER = _BATCH // _NW


def _sc_body(u_hbm, v_hbm, table_hbm, out_hbm,
             uidx_v, vidx_v, ublk, vblk, out_v, sem_u, sem_v):
    wid = lax.axis_index("s") * _NC + lax.axis_index("c")
    base = wid * _N_PER

    pltpu.sync_copy(u_hbm.at[pl.ds(base, _N_PER)], uidx_v)
    pltpu.sync_copy(v_hbm.at[pl.ds(base, _N_PER)], vidx_v)

    lane = lax.iota(jnp.int32, _L)

    def fire(g, b):
        sl = pl.ds(g * _L, _L)
        ubase = jnp.bitwise_and(uidx_v[sl], ~7)
        vbase = jnp.bitwise_and(vidx_v[sl], ~7)
        for k in range(_L):
            ub = pl.multiple_of(ubase[k], 8)
            vb = pl.multiple_of(vbase[k], 8)
            pltpu.async_copy(table_hbm.at[pl.ds(ub, 8), :],
                             ublk.at[b, k], sem_u.at[b])
            pltpu.async_copy(table_hbm.at[pl.ds(vb, 8), :],
                             vblk.at[b, k], sem_v.at[b])

    def drain(b):
        for k in range(_L):
            pltpu.make_async_copy(table_hbm.at[pl.ds(0, 8), :],
                                  ublk.at[b, k], sem_u.at[b]).wait()
            pltpu.make_async_copy(table_hbm.at[pl.ds(0, 8), :],
                                  vblk.at[b, k], sem_v.at[b]).wait()

    def compute(g, b):
        sl = pl.ds(g * _L, _L)
        uoff = jnp.bitwise_and(uidx_v[sl], 7)
        voff = jnp.bitwise_and(vidx_v[sl], 7)
        acc = jnp.zeros((_L,), jnp.float32)
        for j in range(_EMB):
            jc = jnp.full((_L,), j, jnp.int32)
            uu = plsc.load_gather(ublk.at[b], [lane, uoff, jc])
            vv = plsc.load_gather(vblk.at[b], [lane, voff, jc])
            acc = acc + uu * vv
        out_v[sl] = 1.0 / (1.0 + jnp.exp(-acc))

    ngroups = _N_PER // _L
    fire(jnp.int32(0), jnp.int32(0))
    fire(jnp.int32(1), jnp.int32(1))

    def gbody(g, carry):
        b = lax.rem(g, 3)

        @pl.when(g + 2 < ngroups)
        def _():
            fire(g + 2, lax.rem(g + 2, 3))

        drain(b)
        compute(g, b)
        return carry

    lax.fori_loop(0, ngroups, gbody, 0)

    pltpu.sync_copy(out_v, out_hbm.at[pl.ds(base, _N_PER)])


@jax.jit
def _dist_mult(u, v, emb_weight):
    mesh = plsc.VectorSubcoreMesh(
        core_axis_name="c", subcore_axis_name="s",
        num_cores=_NC, num_subcores=_NS)
    run = pl.kernel(
        _sc_body,
        out_type=jax.ShapeDtypeStruct((_BATCH,), jnp.float32),
        mesh=mesh,
        scratch_types=[
            pltpu.VMEM((_N_PER,), jnp.int32),
            pltpu.VMEM((_N_PER,), jnp.int32),
            pltpu.VMEM((3, _L, 8, _EMB), jnp.float32),
            pltpu.VMEM((3, _L, 8, _EMB), jnp.float32),
            pltpu.VMEM((_N_PER,), jnp.float32),
            pltpu.SemaphoreType.DMA((3,)),
            pltpu.SemaphoreType.DMA((3,)),
        ],
        compiler_params=pltpu.CompilerParams(
            needs_layout_passes=False, use_tc_tiling_on_sc=True),
    )
    return run(u, v, emb_weight)


def kernel(u, v, emb_weight):
    return _dist_mult(u.astype(jnp.int32), v.astype(jnp.int32), emb_weight)


# speedup vs baseline: 1.0194x; 1.0007x over previous
import jax
import jax.numpy as jnp
from jax import lax
from jax.experimental import pallas as pl
from jax.experimental.pallas import tpu as pltpu
from jax.experimental.pallas import tpu_sc as plsc

_BATCH = 16384
_EMB = 64
_L = 16
_NC = 2
_NS = 16
_NW = _NC * _NS
_N_PER = _BATCH // _NW


def _sc_body(u_hbm, v_hbm, table_hbm, out_hbm,
             uidx_v, vidx_v, ublk, vblk, out_v, sem_u, sem_v):
    wid = lax.axis_index("s") * _NC + lax.axis_index("c")
    base = wid * _N_PER

    pltpu.sync_copy(u_hbm.at[pl.ds(base, _N_PER)], uidx_v)
    pltpu.sync_copy(v_hbm.at[pl.ds(base, _N_PER)], vidx_v)

    lane = lax.iota(jnp.int32, _L)

    def fire(g, b):
        sl = pl.ds(g * _L, _L)
        ubase = jnp.bitwise_and(uidx_v[sl], ~7)
        vbase = jnp.bitwise_and(vidx_v[sl], ~7)
        for k in range(_L):
            ub = pl.multiple_of(ubase[k], 8)
            vb = pl.multiple_of(vbase[k], 8)
            pltpu.async_copy(table_hbm.at[pl.ds(ub, 8), :],
                             ublk.at[b, k], sem_u.at[b])
            pltpu.async_copy(table_hbm.at[pl.ds(vb, 8), :],
                             vblk.at[b, k], sem_v.at[b])

    def drain(b):
        for k in range(_L):
            pltpu.make_async_copy(table_hbm.at[pl.ds(0, 8), :],
                                  ublk.at[b, k], sem_u.at[b]).wait()
            pltpu.make_async_copy(table_hbm.at[pl.ds(0, 8), :],
                                  vblk.at[b, k], sem_v.at[b]).wait()

    def compute(g, b):
        sl = pl.ds(g * _L, _L)
        uoff = jnp.bitwise_and(uidx_v[sl], 7)
        voff = jnp.bitwise_and(vidx_v[sl], 7)
        acc = jnp.zeros((_L,), jnp.float32)
        for j in range(_EMB):
            jc = jnp.full((_L,), j, jnp.int32)
            uu = plsc.load_gather(ublk.at[b], [lane, uoff, jc])
            vv = plsc.load_gather(vblk.at[b], [lane, voff, jc])
            acc = acc + uu * vv
        out_v[sl] = 1.0 / (1.0 + jnp.exp(-acc))

    ngroups = _N_PER // _L
    fire(jnp.int32(0), jnp.int32(0))
    fire(jnp.int32(1), jnp.int32(1))

    def gbody(g, carry):
        b = lax.rem(g, 3)

        @pl.when(g + 2 < ngroups)
        def _():
            fire(g + 2, lax.rem(g + 2, 3))

        drain(b)
        compute(g, b)
        return carry

    lax.fori_loop(0, ngroups, gbody, 0)

    pltpu.sync_copy(out_v, out_hbm.at[pl.ds(base, _N_PER)])


@jax.jit
def _dist_mult(u, v, emb_weight):
    mesh = plsc.VectorSubcoreMesh(
        core_axis_name="c", subcore_axis_name="s",
        num_cores=_NC, num_subcores=_NS)
    run = pl.kernel(
        _sc_body,
        out_type=jax.ShapeDtypeStruct((_BATCH,), jnp.float32),
        mesh=mesh,
        scratch_types=[
            pltpu.VMEM((_N_PER,), jnp.int32),
            pltpu.VMEM((_N_PER,), jnp.int32),
            pltpu.VMEM((3, _L, 8, _EMB), jnp.float32),
            pltpu.VMEM((3, _L, 8, _EMB), jnp.float32),
            pltpu.VMEM((_N_PER,), jnp.float32),
            pltpu.SemaphoreType.DMA((3,)),
            pltpu.SemaphoreType.DMA((3,)),
        ],
        compiler_params=pltpu.CompilerParams(
            needs_layout_passes=False, use_tc_tiling_on_sc=True),
    )
    return run(u, v, emb_weight)


def kernel(u, v, emb_weight):
    return _dist_mult(u.astype(jnp.int32), v.astype(jnp.int32), emb_weight)
